# Initial kernel scaffold; baseline (speedup 1.0000x reference)
import jax
import jax.numpy as jnp
from jax.experimental import pallas as pl


def kernel(x, edge_index, batch, B, Pi):
    raise NotImplementedError("write your pallas kernel here")


# speedup vs baseline: 116.8623x; 116.8623x over previous
import functools

import jax
import jax.numpy as jnp
from jax import lax
from jax.experimental import pallas as pl
from jax.experimental.pallas import tpu as pltpu
from jax.experimental.pallas import tpu_sc as plsc

C = 32
M = 256
N_GEN = 16
N_GRAPHS = 512
NC = 2
NS = 16
LANES = 16
TABLE_COLS = 512


def _table_body(bt_ref, pi_ref, out_ref):
    bt = bt_ref[...]
    bmax = jnp.max(bt, axis=2, keepdims=True)
    be = jnp.exp(bt - bmax)
    sm_b = be / jnp.sum(be, axis=2, keepdims=True)
    pi = pi_ref[...]
    pmax = jnp.max(pi, axis=0, keepdims=True)
    pe = jnp.exp(pi - pmax)
    sm_pi = pe / jnp.sum(pe, axis=0, keepdims=True)
    num = sm_pi[:, :, None] * sm_b
    denom = jnp.sum(num, axis=0)
    plog = jnp.sum(num * jnp.log(num), axis=0)
    likt = plog / denom
    out_ref[:, :M] = -likt
    out_ref[:, M:] = jnp.zeros((N_GEN, TABLE_COLS - M), jnp.float32)


def _make_table(b_t, pi):
    return pl.pallas_call(
        _table_body,
        out_shape=jax.ShapeDtypeStruct((N_GEN, TABLE_COLS), jnp.float32),
    )(b_t, pi)


def _sc_body(chunk, n_vregs, table_hbm, x_hbm, b_hbm, bn_hbm, out_hbm,
             table_v, x_v, b_v, bn_v, acc_v, buf_v, sum_v, shared):
    cid = lax.axis_index("c")
    sid = lax.axis_index("s")
    wid = sid * NC + cid
    base = wid * chunk

    pltpu.sync_copy(table_hbm, table_v)
    pltpu.sync_copy(x_hbm.at[pl.ds(base, chunk)], x_v)
    pltpu.sync_copy(b_hbm.at[pl.ds(base, chunk)], b_v)
    pltpu.sync_copy(bn_hbm.at[pl.ds(base, chunk)], bn_v)

    zeros16 = jnp.zeros((LANES,), jnp.float32)

    def zero_body(i, carry):
        acc_v[pl.ds(i * LANES, LANES)] = zeros16
        return carry

    lax.fori_loop(0, N_GRAPHS, zero_body, 0)

    lane15 = lax.iota(jnp.int32, LANES) == (LANES - 1)

    def vreg_body(i, carry):
        off = i * LANES
        xv = x_v[pl.ds(off, LANES)]
        bv = b_v[pl.ds(off, LANES)]
        bnv = bn_v[pl.ds(off, LANES)]
        boundary = bv != bnv
        m_last = jnp.logical_or(boundary, lane15)
        m_next = jnp.logical_and(boundary, jnp.logical_not(lane15))
        bv16 = bv * N_GEN
        bn16 = bnv * N_GEN
        for g in range(N_GEN):
            gsplat = jnp.full((LANES,), g, jnp.int32)
            vals = plsc.load_gather(table_v, [gsplat, xv])
            c = plsc.cumsum(vals)
            plsc.addupdate_scatter(acc_v, [bv16 + g], c, mask=m_last)
            plsc.addupdate_scatter(acc_v, [bn16 + g], -c, mask=m_next)
        return carry

    lax.fori_loop(0, n_vregs, vreg_body, 0)

    plsc.subcore_barrier()
    pltpu.sync_copy(acc_v, shared.at[sid])
    plsc.subcore_barrier()

    span = N_GRAPHS * N_GEN // NS
    for s in range(NS):
        pltpu.sync_copy(shared.at[s, pl.ds(sid * span, span)], buf_v)
        for r in range(span // LANES):
            sl = pl.ds(r * LANES, LANES)
            if s == 0:
                sum_v[sl] = buf_v[sl]
            else:
                sum_v[sl] = sum_v[sl] + buf_v[sl]
    pltpu.sync_copy(sum_v, out_hbm.at[cid, pl.ds(sid * span, span)])


def _make_sc(npad):
    chunk = npad // (NC * NS)
    n_vregs = chunk // LANES
    span = N_GRAPHS * N_GEN // NS
    mesh = plsc.VectorSubcoreMesh(core_axis_name="c", subcore_axis_name="s")
    return pl.kernel(
        functools.partial(_sc_body, chunk, n_vregs),
        out_type=jax.ShapeDtypeStruct((NC, N_GRAPHS * N_GEN), jnp.float32),
        mesh=mesh,
        scratch_types=[
            pltpu.VMEM((N_GEN, TABLE_COLS), jnp.float32),
            pltpu.VMEM((chunk,), jnp.int32),
            pltpu.VMEM((chunk,), jnp.int32),
            pltpu.VMEM((chunk,), jnp.int32),
            pltpu.VMEM((N_GRAPHS * N_GEN,), jnp.float32),
            pltpu.VMEM((span,), jnp.float32),
            pltpu.VMEM((span,), jnp.float32),
            pltpu.VMEM_SHARED((NS, N_GRAPHS * N_GEN), jnp.float32),
        ],
        compiler_params=pltpu.CompilerParams(
            use_tc_tiling_on_sc=False, needs_layout_passes=False),
    )


@jax.jit
def kernel(x, edge_index, batch, B, Pi):
    del edge_index
    n = x.shape[0]
    per_tile = ((n + NC * NS * LANES - 1) // (NC * NS * LANES)) * LANES
    npad = per_tile * NC * NS
    pad = npad - n
    x_pad = jnp.concatenate([x, jnp.full((pad,), M, jnp.int32)])
    b_pad = jnp.concatenate([batch, jnp.full((pad,), batch[-1], jnp.int32)])
    bn_pad = jnp.concatenate([b_pad[1:], b_pad[-1:]])

    table = _make_table(jnp.transpose(B, (0, 2, 1)), Pi)
    partials = _make_sc(npad)(table, x_pad, b_pad, bn_pad)
    out = (partials[0] + partials[1]).reshape(N_GRAPHS, 1, N_GEN)
    return out


# speedup vs baseline: 163.0458x; 1.3952x over previous
import functools

import jax
import jax.numpy as jnp
from jax import lax
from jax.experimental import pallas as pl
from jax.experimental.pallas import tpu as pltpu
from jax.experimental.pallas import tpu_sc as plsc

C = 32
M = 256
N_GEN = 16
N_GRAPHS = 512
NC = 2
NS = 16
LANES = 16
TABLE_ROWS = 512
BLK = 128
NBLK = 25
CHUNK = BLK * NBLK


def _table_body(bt_ref, pi_ref, out_ref):
    bt = bt_ref[...]
    bmax = jnp.max(bt, axis=2, keepdims=True)
    be = jnp.exp(bt - bmax)
    sm_b = be / jnp.sum(be, axis=2, keepdims=True)
    pi = pi_ref[...]
    pmax = jnp.max(pi, axis=0, keepdims=True)
    pe = jnp.exp(pi - pmax)
    sm_pi = pe / jnp.sum(pe, axis=0, keepdims=True)
    num = sm_pi[:, :, None] * sm_b
    denom = jnp.sum(num, axis=0)
    plog = jnp.sum(num * jnp.log(num), axis=0)
    likt = plog / denom
    out_ref[:M, :] = -likt.T
    out_ref[M:, :] = jnp.zeros((TABLE_ROWS - M, N_GEN), jnp.float32)


def _make_table(b_t, pi):
    return pl.pallas_call(
        _table_body,
        out_shape=jax.ShapeDtypeStruct((TABLE_ROWS, N_GEN), jnp.float32),
    )(b_t, pi)


def _sc_body(table_hbm, x_hbm, b_hbm, out_hbm,
             x_v, b_v, emis_v, acc_v, buf_v, sum_v, shared, sem):
    cid = lax.axis_index("c")
    sid = lax.axis_index("s")
    wid = sid * NC + cid
    base = wid * CHUNK

    pltpu.sync_copy(x_hbm.at[wid], x_v)
    pltpu.sync_copy(b_hbm.at[pl.ds(base, CHUNK)], b_v)

    copies = []
    for j in range(NBLK):
        copies.append(pltpu.async_copy(
            table_hbm.at[x_v.at[j]], emis_v.at[pl.ds(j * BLK, BLK)], sem))

    zeros16 = jnp.zeros((LANES,), jnp.float32)

    @plsc.parallel_loop(0, N_GRAPHS)
    def _(i):
        acc_v[pl.ds(i * LANES, LANES)] = zeros16

    for cp in copies:
        cp.wait()

    iota = lax.iota(jnp.int32, LANES)
    dnums = lax.GatherDimensionNumbers(
        offset_dims=(), collapsed_slice_dims=(0,), start_index_map=(0,))

    @plsc.parallel_loop(0, CHUNK // LANES)
    def _(g):
        off = g * LANES
        bv16 = b_v[pl.ds(off, LANES)] * N_GEN
        rows = [emis_v[off + nn, :] for nn in range(LANES)]
        idxs = []
        for nn in range(LANES):
            bsp = lax.gather(
                bv16, jnp.full((LANES, 1), nn, jnp.int32), dnums, (1,),
                mode=lax.GatherScatterMode.PROMISE_IN_BOUNDS)
            idxs.append(bsp + iota)
        for nn in range(LANES):
            plsc.addupdate_scatter(acc_v, [idxs[nn]], rows[nn])

    plsc.subcore_barrier()
    pltpu.sync_copy(acc_v, shared.at[sid])
    plsc.subcore_barrier()

    span = N_GRAPHS * N_GEN // NS
    for s in range(NS):
        pltpu.sync_copy(shared.at[s, pl.ds(sid * span, span)], buf_v)
        for r in range(span // LANES):
            sl = pl.ds(r * LANES, LANES)
            if s == 0:
                sum_v[sl] = buf_v[sl]
            else:
                sum_v[sl] = sum_v[sl] + buf_v[sl]
    pltpu.sync_copy(sum_v, out_hbm.at[cid, pl.ds(sid * span, span)])


def _make_sc():
    mesh = plsc.VectorSubcoreMesh(core_axis_name="c", subcore_axis_name="s")
    span = N_GRAPHS * N_GEN // NS
    return pl.kernel(
        _sc_body,
        out_type=jax.ShapeDtypeStruct((NC, N_GRAPHS * N_GEN), jnp.float32),
        mesh=mesh,
        scratch_types=[
            pltpu.VMEM((NBLK, BLK), jnp.int32),
            pltpu.VMEM((CHUNK,), jnp.int32),
            pltpu.VMEM((CHUNK, N_GEN), jnp.float32),
            pltpu.VMEM((N_GRAPHS * N_GEN,), jnp.float32),
            pltpu.VMEM((span,), jnp.float32),
            pltpu.VMEM((span,), jnp.float32),
            pltpu.VMEM_SHARED((NS, N_GRAPHS * N_GEN), jnp.float32),
            pltpu.SemaphoreType.DMA,
        ],
        compiler_params=pltpu.CompilerParams(
            use_tc_tiling_on_sc=False, needs_layout_passes=False),
    )


@jax.jit
def kernel(x, edge_index, batch, B, Pi):
    del edge_index
    n = x.shape[0]
    npad = NC * NS * CHUNK
    pad = npad - n
    x_pad = jnp.concatenate([x, jnp.full((pad,), M, jnp.int32)])
    b_pad = jnp.concatenate([batch, jnp.full((pad,), batch[-1], jnp.int32)])

    table = _make_table(jnp.transpose(B, (0, 2, 1)), Pi)
    partials = _make_sc()(table, x_pad.reshape(NC * NS, NBLK, BLK), b_pad)
    out = (partials[0] + partials[1]).reshape(N_GRAPHS, 1, N_GEN)
    return out


# speedup vs baseline: 217.8966x; 1.3364x over previous
import jax
import jax.numpy as jnp
from jax import lax
from jax.experimental import pallas as pl
from jax.experimental.pallas import tpu as pltpu
from jax.experimental.pallas import tpu_sc as plsc

C = 32
M = 256
N_GEN = 16
N_GRAPHS = 512
NS = 16
LANES = 16
TABLE_ROWS = 512
CHUNK = 6400


def _table_body(bt_ref, pi_ref, out_ref):
    bt = bt_ref[...]
    bmax = jnp.max(bt, axis=2, keepdims=True)
    be = jnp.exp(bt - bmax)
    sm_b = be / jnp.sum(be, axis=2, keepdims=True)
    pi = pi_ref[...]
    pmax = jnp.max(pi, axis=0, keepdims=True)
    pe = jnp.exp(pi - pmax)
    sm_pi = pe / jnp.sum(pe, axis=0, keepdims=True)
    num = sm_pi[:, :, None] * sm_b
    denom = jnp.sum(num, axis=0)
    plog = jnp.sum(num * jnp.log(num), axis=0)
    likt = plog / denom
    out_ref[:M, :] = -likt.T
    out_ref[M:, :] = jnp.zeros((TABLE_ROWS - M, N_GEN), jnp.float32)


def _make_table(b_t, pi):
    return pl.pallas_call(
        _table_body,
        out_shape=jax.ShapeDtypeStruct((TABLE_ROWS, N_GEN), jnp.float32),
    )(b_t, pi)


def _sc_body(table_hbm, x_hbm, b_hbm, out_hbm,
             table_v, x_v, b_v, acc_v, buf_v, sum_v, shared):
    sid = lax.axis_index("s")
    base = sid * CHUNK

    pltpu.sync_copy(table_hbm, table_v)
    pltpu.sync_copy(x_hbm.at[pl.ds(base, CHUNK)], x_v)
    pltpu.sync_copy(b_hbm.at[pl.ds(base, CHUNK)], b_v)

    zeros16 = jnp.zeros((LANES,), jnp.float32)

    @plsc.parallel_loop(0, N_GRAPHS, unroll=4)
    def _(i):
        acc_v[pl.ds(i * LANES, LANES)] = zeros16

    iota = lax.iota(jnp.int32, LANES)
    dnums = lax.GatherDimensionNumbers(
        offset_dims=(), collapsed_slice_dims=(0,), start_index_map=(0,))

    def _splat(vec, nn):
        return lax.gather(
            vec, jnp.full((LANES, 1), nn, jnp.int32), dnums, (1,),
            mode=lax.GatherScatterMode.PROMISE_IN_BOUNDS)

    @plsc.parallel_loop(0, CHUNK // LANES)
    def _(g):
        off = g * LANES
        xv16 = x_v[pl.ds(off, LANES)] * N_GEN
        bv16 = b_v[pl.ds(off, LANES)] * N_GEN
        for nn in range(LANES):
            row = plsc.load_gather(table_v, [_splat(xv16, nn) + iota])
            plsc.addupdate_scatter(
                acc_v, [_splat(bv16, nn) + iota], row)

    plsc.subcore_barrier()
    pltpu.sync_copy(acc_v, shared.at[sid])
    plsc.subcore_barrier()

    span = N_GRAPHS * N_GEN // NS
    for s in range(NS):
        pltpu.sync_copy(shared.at[s, pl.ds(sid * span, span)], buf_v)
        for r in range(span // LANES):
            sl = pl.ds(r * LANES, LANES)
            if s == 0:
                sum_v[sl] = buf_v[sl]
            else:
                sum_v[sl] = sum_v[sl] + buf_v[sl]
    pltpu.sync_copy(sum_v, out_hbm.at[pl.ds(sid * span, span)])


def _make_sc():
    mesh = plsc.VectorSubcoreMesh(
        core_axis_name="c", subcore_axis_name="s", num_cores=1)
    span = N_GRAPHS * N_GEN // NS
    return pl.kernel(
        _sc_body,
        out_type=jax.ShapeDtypeStruct((N_GRAPHS * N_GEN,), jnp.float32),
        mesh=mesh,
        scratch_types=[
            pltpu.VMEM((TABLE_ROWS * N_GEN,), jnp.float32),
            pltpu.VMEM((CHUNK,), jnp.int32),
            pltpu.VMEM((CHUNK,), jnp.int32),
            pltpu.VMEM((N_GRAPHS * N_GEN,), jnp.float32),
            pltpu.VMEM((span,), jnp.float32),
            pltpu.VMEM((span,), jnp.float32),
            pltpu.VMEM_SHARED((NS, N_GRAPHS * N_GEN), jnp.float32),
        ],
        compiler_params=pltpu.CompilerParams(
            use_tc_tiling_on_sc=False, needs_layout_passes=False),
    )


@jax.jit
def kernel(x, edge_index, batch, B, Pi):
    del edge_index
    n = x.shape[0]
    npad = NS * CHUNK
    pad = npad - n
    x_pad = jnp.concatenate([x, jnp.full((pad,), M, jnp.int32)])
    b_pad = jnp.concatenate([batch, jnp.full((pad,), batch[-1], jnp.int32)])

    table = _make_table(jnp.transpose(B, (0, 2, 1)), Pi)
    out = _make_sc()(table.reshape(-1), x_pad, b_pad)
    return out.reshape(N_GRAPHS, 1, N_GEN)
